# baseline (speedup 1.0000x reference)
import functools

import jax
import jax.numpy as jnp
from jax import lax
from jax.experimental import pallas as pl
from jax.experimental.pallas import tpu as pltpu
from jax.experimental.pallas import tpu_sc as plsc

NUM_GROUPS = 8
NUM_CODES = 1024
GROUP_DIM = 32
COMMITMENT_COST = 0.25

TILE_N = 512

_SC_INFO = plsc.get_sparse_core_info()
_NC, _NS = _SC_INFO.num_cores, _SC_INFO.num_subcores
_NW = _NC * _NS
_CHUNK = 1024


def _vq_tc_kernel(x_ref, xsq_ref, embt_ref, esq_ref, iotaf_ref,
                  idx_ref, fidx_ref, sse_ref, dot2_ref):
    tn = x_ref.shape[0]
    for g in range(NUM_GROUPS):
        xg = x_ref[:, g * GROUP_DIM:(g + 1) * GROUP_DIM]
        dot2_ref[:, g * NUM_CODES:(g + 1) * NUM_CODES] = jax.lax.dot_general(
            xg, embt_ref[g], (((1,), (0,)), ((), ())),
            preferred_element_type=jnp.float32)

    iota_f = iotaf_ref[0]
    goff = jax.lax.broadcasted_iota(jnp.int32, (8, NUM_GROUPS), 1) * NUM_CODES

    def body(i, msum):
        r = i * 8
        idx_cols = []
        for g in range(NUM_GROUPS):
            s = dot2_ref[pl.ds(r, 8), g * NUM_CODES:(g + 1) * NUM_CODES]
            a = xsq_ref[pl.ds(r, 8), g:g + 1] + esq_ref[g][None, :]
            dist = a - s
            mind = jnp.min(dist, axis=1, keepdims=True)
            msum = msum + mind
            idx_f = jnp.min(jnp.where(dist == mind, iota_f[None, :],
                                      jnp.float32(NUM_CODES)),
                            axis=1)
            idx_cols.append(idx_f.astype(jnp.int32))
        idx_blk = jnp.stack(idx_cols, axis=1)
        idx_ref[pl.ds(r, 8), :] = idx_blk
        fidx_ref[pl.ds(r, 8), :] = idx_blk + goff
        return msum

    msum = jax.lax.fori_loop(0, tn // 8, body,
                             jnp.zeros((8, 1), jnp.float32), unroll=8)
    psum = jnp.sum(msum)

    @pl.when(pl.program_id(0) == 0)
    def _init():
        sse_ref[0, 0] = jnp.float32(0.0)

    sse_ref[0, 0] += psum


@functools.partial(
    pl.kernel,
    mesh=plsc.VectorSubcoreMesh(core_axis_name="c", subcore_axis_name="s"),
    compiler_params=pltpu.CompilerParams(use_tc_tiling_on_sc=False),
    out_type=jax.ShapeDtypeStruct((16384 * NUM_GROUPS, GROUP_DIM),
                                  jnp.float32),
    scratch_types=[
        pltpu.VMEM((_CHUNK,), jnp.int32),
        pltpu.VMEM((_CHUNK, GROUP_DIM), jnp.float32),
        pltpu.SemaphoreType.DMA,
    ],
)
def _sc_gather(table_hbm, fidx_hbm, out_hbm, idx_v, rows_v, sem):
    b_total = out_hbm.shape[0]
    b_per_w = b_total // _NW
    wid = lax.axis_index("s") * _NC + lax.axis_index("c")
    for c in range(b_per_w // _CHUNK):
        base = wid * b_per_w + c * _CHUNK
        pltpu.sync_copy(fidx_hbm.at[pl.ds(base, _CHUNK)], idx_v)
        pltpu.async_copy(table_hbm.at[idx_v], rows_v, sem).wait()
        pltpu.sync_copy(rows_v, out_hbm.at[pl.ds(base, _CHUNK)])


@jax.jit
def kernel(inputs, embedding):
    n = inputs.shape[0]
    x3 = inputs.reshape(n, NUM_GROUPS, GROUP_DIM)
    inputs_sq = jnp.sum(x3 ** 2, axis=2)
    embed_sq = jnp.sum(embedding ** 2, axis=2)
    emb_t = 2.0 * jnp.transpose(embedding, (0, 2, 1))

    grid = (n // TILE_N,)
    idx, fidx, sse = pl.pallas_call(
        _vq_tc_kernel,
        grid=grid,
        in_specs=[
            pl.BlockSpec((TILE_N, NUM_GROUPS * GROUP_DIM), lambda i: (i, 0)),
            pl.BlockSpec((TILE_N, NUM_GROUPS), lambda i: (i, 0)),
            pl.BlockSpec((NUM_GROUPS, GROUP_DIM, NUM_CODES),
                         lambda i: (0, 0, 0)),
            pl.BlockSpec((NUM_GROUPS, NUM_CODES), lambda i: (0, 0)),
            pl.BlockSpec((1, NUM_CODES), lambda i: (0, 0)),
        ],
        out_specs=[
            pl.BlockSpec((TILE_N, NUM_GROUPS), lambda i: (i, 0)),
            pl.BlockSpec((TILE_N, NUM_GROUPS), lambda i: (i, 0)),
            pl.BlockSpec((1, 1), lambda i: (0, 0),
                         memory_space=pltpu.SMEM),
        ],
        out_shape=[
            jax.ShapeDtypeStruct((n, NUM_GROUPS), jnp.int32),
            jax.ShapeDtypeStruct((n, NUM_GROUPS), jnp.int32),
            jax.ShapeDtypeStruct((1, 1), jnp.float32),
        ],
        scratch_shapes=[
            pltpu.VMEM((TILE_N, NUM_GROUPS * NUM_CODES), jnp.float32),
        ],
    )(inputs, inputs_sq, emb_t, embed_sq,
      jnp.arange(NUM_CODES, dtype=jnp.float32).reshape(1, NUM_CODES))

    table = embedding.reshape(NUM_GROUPS * NUM_CODES, GROUP_DIM)
    rows = _sc_gather(table, fidx.reshape(n * NUM_GROUPS))
    q = rows.reshape(n, NUM_GROUPS * GROUP_DIM)

    total = jnp.float32(n * NUM_GROUPS * GROUP_DIM)
    codebook_loss = sse[0, 0] / total
    commit_loss = codebook_loss
    vq_loss = codebook_loss + COMMITMENT_COST * commit_loss
    indices = idx.astype(jnp.int64)
    return (q, indices, vq_loss, codebook_loss, commit_loss)


# speedup vs baseline: 5.4834x; 5.4834x over previous
import functools

import jax
import jax.numpy as jnp
from jax import lax
from jax.experimental import pallas as pl
from jax.experimental.pallas import tpu as pltpu
from jax.experimental.pallas import tpu_sc as plsc

NUM_GROUPS = 8
NUM_CODES = 1024
GROUP_DIM = 32
COMMITMENT_COST = 0.25

TILE_N = 512

_SC_INFO = plsc.get_sparse_core_info()
_NC, _NS = _SC_INFO.num_cores, _SC_INFO.num_subcores
_NW = _NC * _NS
_CHUNK = 1024


def _vq_tc_kernel(x_ref, xsq_ref, embt_ref, esq_ref, iotaf_ref,
                  idx_ref, fidx_ref, sse_ref):
    tn = x_ref.shape[0]
    msum = None
    idx_cols = []
    fidx_cols = []
    for g in range(NUM_GROUPS):
        xg = x_ref[:, g * GROUP_DIM:(g + 1) * GROUP_DIM]
        embt_g = embt_ref[g]
        dot2 = jax.lax.dot_general(
            xg, embt_g, (((1,), (0,)), ((), ())),
            preferred_element_type=jnp.float32)
        a = xsq_ref[:, g:g + 1] + esq_ref[g][None, :]
        dist = a - dot2
        mind = jnp.min(dist, axis=1)
        msum = mind if msum is None else msum + mind
        idx_f = jnp.min(jnp.where(dist == mind[:, None], iotaf_ref[0],
                                  jnp.float32(NUM_CODES)),
                        axis=1)
        idx = idx_f.astype(jnp.int32)
        idx_cols.append(idx)
        fidx_cols.append(idx + g * NUM_CODES)
    idx_ref[...] = jnp.stack(idx_cols, axis=1)
    fidx_ref[...] = jnp.stack(fidx_cols, axis=1)
    psum = jnp.sum(msum)

    @pl.when(pl.program_id(0) == 0)
    def _init():
        sse_ref[0, 0] = jnp.float32(0.0)

    sse_ref[0, 0] += psum


@functools.partial(
    pl.kernel,
    mesh=plsc.VectorSubcoreMesh(core_axis_name="c", subcore_axis_name="s"),
    compiler_params=pltpu.CompilerParams(use_tc_tiling_on_sc=False),
    out_type=jax.ShapeDtypeStruct((16384 * NUM_GROUPS, GROUP_DIM),
                                  jnp.float32),
    scratch_types=[
        pltpu.VMEM((_CHUNK,), jnp.int32),
        pltpu.VMEM((_CHUNK, GROUP_DIM), jnp.float32),
        pltpu.SemaphoreType.DMA,
    ],
)
def _sc_gather(table_hbm, fidx_hbm, out_hbm, idx_v, rows_v, sem):
    b_total = out_hbm.shape[0]
    b_per_w = b_total // _NW
    wid = lax.axis_index("s") * _NC + lax.axis_index("c")
    for c in range(b_per_w // _CHUNK):
        base = wid * b_per_w + c * _CHUNK
        pltpu.sync_copy(fidx_hbm.at[pl.ds(base, _CHUNK)], idx_v)
        pltpu.async_copy(table_hbm.at[idx_v], rows_v, sem).wait()
        pltpu.sync_copy(rows_v, out_hbm.at[pl.ds(base, _CHUNK)])


@jax.jit
def kernel(inputs, embedding):
    n = inputs.shape[0]
    x3 = inputs.reshape(n, NUM_GROUPS, GROUP_DIM)
    inputs_sq = jnp.sum(x3 ** 2, axis=2)
    embed_sq = jnp.sum(embedding ** 2, axis=2)
    emb_t = 2.0 * jnp.transpose(embedding, (0, 2, 1))

    grid = (n // TILE_N,)
    idx, fidx, sse = pl.pallas_call(
        _vq_tc_kernel,
        grid=grid,
        in_specs=[
            pl.BlockSpec((TILE_N, NUM_GROUPS * GROUP_DIM), lambda i: (i, 0)),
            pl.BlockSpec((TILE_N, NUM_GROUPS), lambda i: (i, 0)),
            pl.BlockSpec((NUM_GROUPS, GROUP_DIM, NUM_CODES),
                         lambda i: (0, 0, 0)),
            pl.BlockSpec((NUM_GROUPS, NUM_CODES), lambda i: (0, 0)),
            pl.BlockSpec((1, NUM_CODES), lambda i: (0, 0)),
        ],
        out_specs=[
            pl.BlockSpec((TILE_N, NUM_GROUPS), lambda i: (i, 0)),
            pl.BlockSpec((TILE_N, NUM_GROUPS), lambda i: (i, 0)),
            pl.BlockSpec((1, 1), lambda i: (0, 0),
                         memory_space=pltpu.SMEM),
        ],
        out_shape=[
            jax.ShapeDtypeStruct((n, NUM_GROUPS), jnp.int32),
            jax.ShapeDtypeStruct((n, NUM_GROUPS), jnp.int32),
            jax.ShapeDtypeStruct((1, 1), jnp.float32),
        ],
    )(inputs, inputs_sq, emb_t, embed_sq,
      jnp.arange(NUM_CODES, dtype=jnp.float32).reshape(1, NUM_CODES))

    table = embedding.reshape(NUM_GROUPS * NUM_CODES, GROUP_DIM)
    rows = _sc_gather(table, fidx.reshape(n * NUM_GROUPS))
    q = rows.reshape(n, NUM_GROUPS * GROUP_DIM)

    total = jnp.float32(n * NUM_GROUPS * GROUP_DIM)
    codebook_loss = sse[0, 0] / total
    commit_loss = codebook_loss
    vq_loss = codebook_loss + COMMITMENT_COST * commit_loss
    indices = idx.astype(jnp.int64)
    return (q, indices, vq_loss, codebook_loss, commit_loss)


# speedup vs baseline: 6.5723x; 1.1986x over previous
import functools

import jax
import jax.numpy as jnp
from jax import lax
from jax.experimental import pallas as pl
from jax.experimental.pallas import tpu as pltpu
from jax.experimental.pallas import tpu_sc as plsc

NUM_GROUPS = 8
NUM_CODES = 1024
GROUP_DIM = 32
COMMITMENT_COST = 0.25

TILE_N = 2048

_SC_INFO = plsc.get_sparse_core_info()
_NC, _NS = _SC_INFO.num_cores, _SC_INFO.num_subcores
_NW = _NC * _NS
_CHUNK = 1024


def _vq_tc_kernel(x_ref, xsq_ref, embt_ref, esq_ref, iotaf_ref,
                  idx_ref, fidx_ref, sse_ref):
    tn = x_ref.shape[0]
    msum = None
    idx_cols = []
    fidx_cols = []
    for g in range(NUM_GROUPS):
        xg = x_ref[:, g * GROUP_DIM:(g + 1) * GROUP_DIM]
        embt_g = embt_ref[g]
        dot2 = jax.lax.dot_general(
            xg, embt_g, (((1,), (0,)), ((), ())),
            preferred_element_type=jnp.float32)
        a = xsq_ref[:, g:g + 1] + esq_ref[g][None, :]
        dist = a - dot2
        mind = jnp.min(dist, axis=1)
        msum = mind if msum is None else msum + mind
        idx_f = jnp.min(jnp.where(dist == mind[:, None], iotaf_ref[0],
                                  jnp.float32(NUM_CODES)),
                        axis=1)
        idx = idx_f.astype(jnp.int32)
        idx_cols.append(idx)
        fidx_cols.append(idx + g * NUM_CODES)
    idx_ref[...] = jnp.stack(idx_cols, axis=1)
    fidx_ref[...] = jnp.stack(fidx_cols, axis=1)
    psum = jnp.sum(msum)

    @pl.when(pl.program_id(0) == 0)
    def _init():
        sse_ref[0, 0] = jnp.float32(0.0)

    sse_ref[0, 0] += psum


@functools.partial(
    pl.kernel,
    mesh=plsc.VectorSubcoreMesh(core_axis_name="c", subcore_axis_name="s"),
    compiler_params=pltpu.CompilerParams(use_tc_tiling_on_sc=False),
    out_type=jax.ShapeDtypeStruct((16384 * NUM_GROUPS, GROUP_DIM),
                                  jnp.float32),
    scratch_types=[
        pltpu.VMEM((_CHUNK,), jnp.int32),
        pltpu.VMEM((_CHUNK, GROUP_DIM), jnp.float32),
        pltpu.SemaphoreType.DMA,
    ],
)
def _sc_gather(table_hbm, fidx_hbm, out_hbm, idx_v, rows_v, sem):
    b_total = out_hbm.shape[0]
    b_per_w = b_total // _NW
    wid = lax.axis_index("s") * _NC + lax.axis_index("c")
    for c in range(b_per_w // _CHUNK):
        base = wid * b_per_w + c * _CHUNK
        pltpu.sync_copy(fidx_hbm.at[pl.ds(base, _CHUNK)], idx_v)
        pltpu.async_copy(table_hbm.at[idx_v], rows_v, sem).wait()
        pltpu.sync_copy(rows_v, out_hbm.at[pl.ds(base, _CHUNK)])


@jax.jit
def kernel(inputs, embedding):
    n = inputs.shape[0]
    x3 = inputs.reshape(n, NUM_GROUPS, GROUP_DIM)
    inputs_sq = jnp.sum(x3 ** 2, axis=2)
    embed_sq = jnp.sum(embedding ** 2, axis=2)
    emb_t = 2.0 * jnp.transpose(embedding, (0, 2, 1))

    grid = (n // TILE_N,)
    idx, fidx, sse = pl.pallas_call(
        _vq_tc_kernel,
        grid=grid,
        in_specs=[
            pl.BlockSpec((TILE_N, NUM_GROUPS * GROUP_DIM), lambda i: (i, 0)),
            pl.BlockSpec((TILE_N, NUM_GROUPS), lambda i: (i, 0)),
            pl.BlockSpec((NUM_GROUPS, GROUP_DIM, NUM_CODES),
                         lambda i: (0, 0, 0)),
            pl.BlockSpec((NUM_GROUPS, NUM_CODES), lambda i: (0, 0)),
            pl.BlockSpec((1, NUM_CODES), lambda i: (0, 0)),
        ],
        out_specs=[
            pl.BlockSpec((TILE_N, NUM_GROUPS), lambda i: (i, 0)),
            pl.BlockSpec((TILE_N, NUM_GROUPS), lambda i: (i, 0)),
            pl.BlockSpec((1, 1), lambda i: (0, 0),
                         memory_space=pltpu.SMEM),
        ],
        out_shape=[
            jax.ShapeDtypeStruct((n, NUM_GROUPS), jnp.int32),
            jax.ShapeDtypeStruct((n, NUM_GROUPS), jnp.int32),
            jax.ShapeDtypeStruct((1, 1), jnp.float32),
        ],
    )(inputs, inputs_sq, emb_t, embed_sq,
      jnp.arange(NUM_CODES, dtype=jnp.float32).reshape(1, NUM_CODES))

    table = embedding.reshape(NUM_GROUPS * NUM_CODES, GROUP_DIM)
    rows = _sc_gather(table, fidx.reshape(n * NUM_GROUPS))
    q = rows.reshape(n, NUM_GROUPS * GROUP_DIM)

    total = jnp.float32(n * NUM_GROUPS * GROUP_DIM)
    codebook_loss = sse[0, 0] / total
    commit_loss = codebook_loss
    vq_loss = codebook_loss + COMMITMENT_COST * commit_loss
    indices = idx.astype(jnp.int64)
    return (q, indices, vq_loss, codebook_loss, commit_loss)
